# Initial kernel scaffold; baseline (speedup 1.0000x reference)
import jax
import jax.numpy as jnp
from jax.experimental import pallas as pl


def kernel(x, edge_index, W_in, b_in, W_d0, b_d0, W_d1, b_d1, W_l1a, b_l1a, W_l1b, b_l1b, W_l2a, b_l2a, W_l2b, b_l2b, W_u1, b_u1, W_u0, b_u0, W_out, b_out):
    raise NotImplementedError("write your pallas kernel here")


# speedup vs baseline: 1.9556x; 1.9556x over previous
import functools

import jax
import jax.numpy as jnp
from jax import lax
from jax.experimental import pallas as pl
from jax.experimental.pallas import tpu as pltpu
from jax.experimental.pallas import tpu_sc as plsc

N = 10000
E = 160000
NC = 2
NS = 16
CHUNK = 128
NCHUNKS = 80
E_PAD = NS * CHUNK * NCHUNKS
ROWS_PER_TILE = 632
NACC = NS * ROWS_PER_TILE
OUT_ROWS = 624

_mesh = plsc.VectorSubcoreMesh(core_axis_name="c", subcore_axis_name="s")


def _edge_body(c, s, src_h, dst_h, buf_src, buf_dst, step_fn):
    base = s * (NCHUNKS * CHUNK)
    coff = c * N

    def chunk_body(j, carry):
        off = base + j * CHUNK
        pltpu.sync_copy(src_h.at[pl.ds(off, CHUNK)], buf_src)
        pltpu.sync_copy(dst_h.at[pl.ds(off, CHUNK)], buf_dst)
        for i in range(CHUNK // 16):
            sl = pl.ds(i * 16, 16)
            buf_dst[sl] = buf_dst[sl] + coff
        step_fn(j)
        return carry

    lax.fori_loop(0, NCHUNKS, chunk_body, 0)


def _drain(c, s, acc, out):
    osl = pl.ds(s * OUT_ROWS, OUT_ROWS)
    pltpu.sync_copy(acc.at[osl], out.at[c, osl])

    @pl.when(s == NS - 1)
    def _():
        tail = pl.ds(NS * OUT_ROWS, N - NS * OUT_ROWS)
        pltpu.sync_copy(acc.at[tail], out.at[c, tail])


def _make_spmv(C2):
    def body(table, src_h, dst_h, zero_h, out, buf_src, buf_dst, rows, acc,
             sem):
        c = lax.axis_index("c")
        s = lax.axis_index("s")
        zsl = pl.ds(s * ROWS_PER_TILE, ROWS_PER_TILE)
        pltpu.sync_copy(zero_h.at[zsl], acc.at[zsl])
        plsc.subcore_barrier()

        def step(j):
            pltpu.async_copy(table.at[buf_dst], rows, sem).wait()
            pltpu.sync_copy(rows, acc.at[buf_src], add=True)

        _edge_body(c, s, src_h, dst_h, buf_src, buf_dst, step)
        plsc.subcore_barrier()
        _drain(c, s, acc, out)

    return pl.kernel(
        body,
        out_type=jax.ShapeDtypeStruct((NC, N, C2), jnp.float32),
        mesh=_mesh,
        compiler_params=pltpu.CompilerParams(use_tc_tiling_on_sc=False),
        scratch_types=[
            pltpu.VMEM((CHUNK,), jnp.int32),
            pltpu.VMEM((CHUNK,), jnp.int32),
            pltpu.VMEM((CHUNK, C2), jnp.float32),
            pltpu.VMEM_SHARED((NACC, C2), jnp.float32),
            pltpu.SemaphoreType.DMA,
        ],
    )


def _make_edge_relu(C2):
    def body(tp, tq, src_h, dst_h, zero_h, out, buf_src, buf_dst, buf_srcg,
             rows_p, rows_q, acc, sem):
        c = lax.axis_index("c")
        s = lax.axis_index("s")
        coff = c * N
        zsl = pl.ds(s * ROWS_PER_TILE, ROWS_PER_TILE)
        pltpu.sync_copy(zero_h.at[zsl], acc.at[zsl])
        plsc.subcore_barrier()

        def step(j):
            for i in range(CHUNK // 16):
                sl = pl.ds(i * 16, 16)
                buf_srcg[sl] = buf_src[sl] + coff
            pltpu.async_copy(tp.at[buf_dst], rows_p, sem).wait()
            pltpu.async_copy(tq.at[buf_srcg], rows_q, sem).wait()

            def row_body(r, carry):
                for i in range(C2 // 16):
                    sl = pl.ds(i * 16, 16)
                    rows_p[r, sl] = jnp.maximum(rows_p[r, sl] + rows_q[r, sl],
                                                0.0)
                return carry

            lax.fori_loop(0, CHUNK, row_body, 0)
            pltpu.sync_copy(rows_p, acc.at[buf_src], add=True)

        _edge_body(c, s, src_h, dst_h, buf_src, buf_dst, step)
        plsc.subcore_barrier()
        _drain(c, s, acc, out)

    return pl.kernel(
        body,
        out_type=jax.ShapeDtypeStruct((NC, N, C2), jnp.float32),
        mesh=_mesh,
        compiler_params=pltpu.CompilerParams(use_tc_tiling_on_sc=False),
        scratch_types=[
            pltpu.VMEM((CHUNK,), jnp.int32),
            pltpu.VMEM((CHUNK,), jnp.int32),
            pltpu.VMEM((CHUNK,), jnp.int32),
            pltpu.VMEM((CHUNK, C2), jnp.float32),
            pltpu.VMEM((CHUNK, C2), jnp.float32),
            pltpu.VMEM_SHARED((NACC, C2), jnp.float32),
            pltpu.SemaphoreType.DMA,
        ],
    )


def _stack2(t):
    c2 = t.shape[1] // 2
    return jnp.concatenate([t[:, :c2], t[:, c2:]], axis=0)


def _unstack(o):
    return jnp.concatenate([o[0], o[1]], axis=-1)


def _spmv(table, src_s, dst_g):
    c2 = table.shape[1] // 2
    zeros = jnp.zeros((NACC, c2), jnp.float32)
    out = _make_spmv(c2)(_stack2(table), src_s, dst_g, zeros)
    return _unstack(out)


def _edge_relu_sum(tp, tq, src_s, dst_g):
    c2 = tp.shape[1] // 2
    zeros = jnp.zeros((NACC, c2), jnp.float32)
    out = _make_edge_relu(c2)(_stack2(tp), _stack2(tq), src_s, dst_g, zeros)
    return _unstack(out)


def _tc(fn, out_shapes, *args):
    shapes = [jax.ShapeDtypeStruct(s, jnp.float32) for s in out_shapes]
    outs = pl.pallas_call(fn, out_shape=shapes)(*args)
    return outs


def _k_in(x_ref, w_ref, b_ref, o_ref):
    o_ref[...] = jnp.maximum(
        jnp.dot(x_ref[...], w_ref[...], preferred_element_type=jnp.float32)
        + b_ref[...], 0.0)


def _k_post1(r_ref, wd_ref, h_ref, ws_ref, b_ref, deg_ref, o_ref):
    t = jnp.dot(h_ref[...], ws_ref[...], preferred_element_type=jnp.float32)
    o_ref[...] = jnp.maximum(
        jnp.dot(r_ref[...], wd_ref[...], preferred_element_type=jnp.float32)
        + deg_ref[...] * (t + b_ref[...]), 0.0)


def _k_pq(h_ref, wd_ref, ba_ref, ws_ref, p_ref, q_ref):
    p_ref[...] = jnp.dot(h_ref[...], wd_ref[...],
                         preferred_element_type=jnp.float32) + ba_ref[...]
    q_ref[...] = jnp.dot(h_ref[...], ws_ref[...],
                         preferred_element_type=jnp.float32)


def _k_stats(p_ref, q_ref, r_ref, di_ref, do_ref, s1_ref, s2_ref):
    p = p_ref[...]
    q = q_ref[...]
    s1_ref[...] = jnp.sum(di_ref[...] * p + do_ref[...] * q, axis=0,
                          keepdims=True)
    s2_ref[...] = jnp.sum(di_ref[...] * p * p + do_ref[...] * q * q
                          + 2.0 * q * r_ref[...], axis=0, keepdims=True)


def _k_scale(p_ref, q_ref, s1_ref, s2_ref, pt_ref, qt_ref):
    mean = s1_ref[...] * (1.0 / E)
    var = s2_ref[...] * (1.0 / E) - mean * mean
    scale = lax.rsqrt(var + 1e-5)
    pt_ref[...] = (p_ref[...] - mean) * scale
    qt_ref[...] = q_ref[...] * scale


def _k_post2_relu(g_ref, wb_ref, bb_ref, deg_ref, o_ref):
    o_ref[...] = jnp.maximum(
        jnp.dot(g_ref[...], wb_ref[...], preferred_element_type=jnp.float32)
        + deg_ref[...] * bb_ref[...], 0.0)


def _k_post2(g_ref, wb_ref, bb_ref, deg_ref, o_ref):
    o_ref[...] = jnp.dot(g_ref[...], wb_ref[...],
                         preferred_element_type=jnp.float32) \
        + deg_ref[...] * bb_ref[...]


def _k_premul(h_ref, wd_ref, ws_ref, b_ref, deg_ref, skip_ref, u_ref, v_ref):
    u_ref[...] = jnp.dot(h_ref[...], wd_ref[...],
                         preferred_element_type=jnp.float32)
    t = jnp.dot(h_ref[...], ws_ref[...], preferred_element_type=jnp.float32)
    v_ref[...] = deg_ref[...] * (t + b_ref[...]) + skip_ref[...]


def _k_fuse_u(ru_ref, v_ref, wd_ref, ws_ref, b_ref, deg_ref, skip_ref,
              u_ref, v2_ref):
    h = jnp.maximum(ru_ref[...] + v_ref[...], 0.0)
    u_ref[...] = jnp.dot(h, wd_ref[...], preferred_element_type=jnp.float32)
    t = jnp.dot(h, ws_ref[...], preferred_element_type=jnp.float32)
    v2_ref[...] = deg_ref[...] * (t + b_ref[...]) + skip_ref[...]


def _k_out(ru_ref, v_ref, w_ref, b_ref, o_ref):
    h = jnp.maximum(ru_ref[...] + v_ref[...], 0.0)
    o_ref[...] = jnp.dot(h, w_ref[...],
                         preferred_element_type=jnp.float32) + b_ref[...]


def kernel(x, edge_index, W_in, b_in, W_d0, b_d0, W_d1, b_d1,
           W_l1a, b_l1a, W_l1b, b_l1b, W_l2a, b_l2a, W_l2b, b_l2b,
           W_u1, b_u1, W_u0, b_u0, W_out, b_out):
    src = edge_index[0].astype(jnp.int32)
    dst = edge_index[1].astype(jnp.int32)
    pad = E_PAD - E
    src_s = jnp.concatenate([src, jnp.full((pad,), N, jnp.int32)])
    dst_g = jnp.concatenate([dst, jnp.zeros((pad,), jnp.int32)])
    dst_s = jnp.concatenate([dst, jnp.full((pad,), N, jnp.int32)])
    src_g = jnp.concatenate([src, jnp.zeros((pad,), jnp.int32)])

    ones_tab = jnp.ones((2 * N, 16), jnp.float32)
    zeros16 = jnp.zeros((NACC, 16), jnp.float32)
    spmv16 = _make_spmv(16)
    deg_out = spmv16(ones_tab, src_s, dst_g, zeros16)[0, :, 0:1]
    deg_in = spmv16(ones_tab, dst_s, src_g, zeros16)[0, :, 0:1]

    H = W_in.shape[1]

    def pl2_block(h, Wa, ba, Wb, bb, relu_out):
        F = h.shape[1]
        P, Q = _tc(_k_pq, [(N, Wa.shape[1])] * 2,
                   h, Wa[:F], ba.reshape(1, -1), Wa[F:])
        R = _spmv(P, src_s, dst_g)
        S1, S2 = _tc(_k_stats, [(1, P.shape[1])] * 2, P, Q, R, deg_in,
                     deg_out)
        Pt, Qt = _tc(_k_scale, [P.shape] * 2, P, Q, S1, S2)
        G = _edge_relu_sum(Pt, Qt, src_s, dst_g)
        post = _k_post2_relu if relu_out else _k_post2
        (h2,) = _tc(post, [(N, Wb.shape[1])], G, Wb, bb.reshape(1, -1),
                    deg_out)
        return h2

    (h0,) = _tc(_k_in, [(N, H)], x, W_in, b_in.reshape(1, -1))
    R0 = _spmv(h0, src_s, dst_g)
    (h1,) = _tc(_k_post1, [(N, 2 * H)], R0, W_d0[:H], h0, W_d0[H:],
                b_d0.reshape(1, -1), deg_out)
    R1 = _spmv(h1, src_s, dst_g)
    (h2,) = _tc(_k_post1, [(N, 4 * H)], R1, W_d1[:2 * H], h1, W_d1[2 * H:],
                b_d1.reshape(1, -1), deg_out)
    h3 = pl2_block(h2, W_l1a, b_l1a, W_l1b, b_l1b, relu_out=True)
    h4 = pl2_block(h3, W_l2a, b_l2a, W_l2b, b_l2b, relu_out=False)
    U1, V1 = _tc(_k_premul, [(N, 2 * H)] * 2, h4, W_u1[:4 * H],
                 W_u1[4 * H:], b_u1.reshape(1, -1), deg_out, h1)
    RU1 = _spmv(U1, src_s, dst_g)
    U0, V0 = _tc(_k_fuse_u, [(N, H)] * 2, RU1, V1, W_u0[:2 * H],
                 W_u0[2 * H:], b_u0.reshape(1, -1), deg_out, h0)
    RU0 = _spmv(U0, src_s, dst_g)
    (out,) = _tc(_k_out, [(N, W_out.shape[1])], RU0, V0, W_out,
                 b_out.reshape(1, -1))
    return out


# speedup vs baseline: 2.7451x; 1.4038x over previous
import jax
import jax.numpy as jnp
from jax import lax
from jax.experimental import pallas as pl
from jax.experimental.pallas import tpu as pltpu
from jax.experimental.pallas import tpu_sc as plsc

N = 10000
E = 160000
NC = 2
NS = 16
E_PAD = 163840
EPT = E_PAD // NS
BLK = 16
ROWS_PER_TILE = 632
NACC = NS * ROWS_PER_TILE
OUT_ROWS = 624

_mesh = plsc.VectorSubcoreMesh(core_axis_name="c", subcore_axis_name="s")
_sc_params = pltpu.CompilerParams(use_tc_tiling_on_sc=False)


def _zero_acc(s, zero_h, acc):
    zsl = pl.ds(s * ROWS_PER_TILE, ROWS_PER_TILE)
    pltpu.sync_copy(zero_h.at[zsl], acc.at[zsl])


def _drain(c, s, acc, out):
    osl = pl.ds(s * OUT_ROWS, OUT_ROWS)
    pltpu.sync_copy(acc.at[osl], out.at[c, osl])

    @pl.when(s == NS - 1)
    def _():
        tail = pl.ds(NS * OUT_ROWS, N - NS * OUT_ROWS)
        pltpu.sync_copy(acc.at[tail], out.at[c, tail])


def _offset_rows(idx_dst, coff, chunk):
    def off_body(r, carry):
        for i in range(chunk // 16):
            sl = pl.ds(i * 16, 16)
            idx_dst[r, sl] = idx_dst[r, sl] + coff
        return carry

    lax.fori_loop(0, BLK, off_body, 0)


def _make_spmv(C2, chunk=128):
    nchunks = EPT // chunk
    nblk = nchunks // BLK

    def body(table, src_h, dst_h, zero_h, out, idx_src, idx_dst, rows0,
             rows1, acc, sem0, sem1):
        c = lax.axis_index("c")
        s = lax.axis_index("s")
        _zero_acc(s, zero_h, acc)
        plsc.subcore_barrier()
        coff = c * N
        rows = (rows0, rows1)
        sems = (sem0, sem1)

        def blk_body(b, carry):
            row0 = s * nchunks + b * BLK
            pltpu.sync_copy(src_h.at[pl.ds(row0, BLK)], idx_src)
            pltpu.sync_copy(dst_h.at[pl.ds(row0, BLK)], idx_dst)
            _offset_rows(idx_dst, coff, chunk)
            pltpu.async_copy(table.at[idx_dst.at[0]], rows0, sem0)

            def pair_body(j2, carry2):
                for k in range(2):
                    j = j2 * 2 + k
                    jn = jnp.minimum(j + 1, BLK - 1)
                    pltpu.make_async_copy(table.at[idx_dst.at[j]], rows[k],
                                          sems[k]).wait()
                    pltpu.async_copy(table.at[idx_dst.at[jn]], rows[1 - k],
                                     sems[1 - k])
                    pltpu.sync_copy(rows[k], acc.at[idx_src.at[j]], add=True)
                return carry2

            lax.fori_loop(0, BLK // 2, pair_body, 0)
            pltpu.make_async_copy(table.at[idx_dst.at[BLK - 1]], rows0,
                                  sems[0]).wait()
            return carry

        lax.fori_loop(0, nblk, blk_body, 0)
        plsc.subcore_barrier()
        _drain(c, s, acc, out)

    return pl.kernel(
        body,
        out_type=jax.ShapeDtypeStruct((NC, N, C2), jnp.float32),
        mesh=_mesh,
        compiler_params=_sc_params,
        scratch_types=[
            pltpu.VMEM((BLK, chunk), jnp.int32),
            pltpu.VMEM((BLK, chunk), jnp.int32),
            pltpu.VMEM((chunk, C2), jnp.float32),
            pltpu.VMEM((chunk, C2), jnp.float32),
            pltpu.VMEM_SHARED((NACC, C2), jnp.float32),
            pltpu.SemaphoreType.DMA,
            pltpu.SemaphoreType.DMA,
        ],
    )


def _make_edge_relu(C2, chunk=64):
    nchunks = EPT // chunk
    nblk = nchunks // BLK

    def body(tp, tq, src_h, dst_h, zero_h, out, idx_src, idx_dst, idx_srcg,
             rp0, rp1, rq0, rq1, acc, sp0, sp1, sq0, sq1):
        c = lax.axis_index("c")
        s = lax.axis_index("s")
        _zero_acc(s, zero_h, acc)
        plsc.subcore_barrier()
        coff = c * N
        rp = (rp0, rp1)
        rq = (rq0, rq1)
        sp = (sp0, sp1)
        sq = (sq0, sq1)

        def blk_body(b, carry):
            row0 = s * nchunks + b * BLK
            pltpu.sync_copy(src_h.at[pl.ds(row0, BLK)], idx_src)
            pltpu.sync_copy(dst_h.at[pl.ds(row0, BLK)], idx_dst)
            _offset_rows(idx_dst, coff, chunk)

            def offg_body(r, carry2):
                for i in range(chunk // 16):
                    sl = pl.ds(i * 16, 16)
                    idx_srcg[r, sl] = idx_src[r, sl] + coff
                return carry2

            lax.fori_loop(0, BLK, offg_body, 0)
            pltpu.async_copy(tp.at[idx_dst.at[0]], rp0, sp0)
            pltpu.async_copy(tq.at[idx_srcg.at[0]], rq0, sq0)

            def pair_body(j2, carry2):
                for k in range(2):
                    j = j2 * 2 + k
                    jn = jnp.minimum(j + 1, BLK - 1)
                    pltpu.make_async_copy(tp.at[idx_dst.at[j]], rp[k],
                                          sp[k]).wait()
                    pltpu.make_async_copy(tq.at[idx_srcg.at[j]], rq[k],
                                          sq[k]).wait()
                    pltpu.async_copy(tp.at[idx_dst.at[jn]], rp[1 - k],
                                     sp[1 - k])
                    pltpu.async_copy(tq.at[idx_srcg.at[jn]], rq[1 - k],
                                     sq[1 - k])

                    def row_body(r, carry3):
                        for i in range(C2 // 16):
                            sl = pl.ds(i * 16, 16)
                            rp[k][r, sl] = jnp.maximum(
                                rp[k][r, sl] + rq[k][r, sl], 0.0)
                        return carry3

                    lax.fori_loop(0, chunk, row_body, 0)
                    pltpu.sync_copy(rp[k], acc.at[idx_src.at[j]], add=True)
                return carry2

            lax.fori_loop(0, BLK // 2, pair_body, 0)
            pltpu.make_async_copy(tp.at[idx_dst.at[BLK - 1]], rp0,
                                  sp[0]).wait()
            pltpu.make_async_copy(tq.at[idx_srcg.at[BLK - 1]], rq0,
                                  sq[0]).wait()
            return carry

        lax.fori_loop(0, nblk, blk_body, 0)
        plsc.subcore_barrier()
        _drain(c, s, acc, out)

    return pl.kernel(
        body,
        out_type=jax.ShapeDtypeStruct((NC, N, C2), jnp.float32),
        mesh=_mesh,
        compiler_params=_sc_params,
        scratch_types=[
            pltpu.VMEM((BLK, chunk), jnp.int32),
            pltpu.VMEM((BLK, chunk), jnp.int32),
            pltpu.VMEM((BLK, chunk), jnp.int32),
            pltpu.VMEM((chunk, C2), jnp.float32),
            pltpu.VMEM((chunk, C2), jnp.float32),
            pltpu.VMEM((chunk, C2), jnp.float32),
            pltpu.VMEM((chunk, C2), jnp.float32),
            pltpu.VMEM_SHARED((NACC, C2), jnp.float32),
            pltpu.SemaphoreType.DMA,
            pltpu.SemaphoreType.DMA,
            pltpu.SemaphoreType.DMA,
            pltpu.SemaphoreType.DMA,
        ],
    )


def _stack2(t):
    c2 = t.shape[1] // 2
    return jnp.concatenate([t[:, :c2], t[:, c2:]], axis=0)


def _unstack(o):
    return jnp.concatenate([o[0], o[1]], axis=-1)


def _spmv(table, src_s, dst_g):
    c2 = table.shape[1] // 2
    zeros = jnp.zeros((NACC, c2), jnp.float32)
    out = _make_spmv(c2)(_stack2(table), src_s, dst_g, zeros)
    return _unstack(out)


def _edge_relu_sum(tp, tq, src_s, dst_g):
    c2 = tp.shape[1] // 2
    zeros = jnp.zeros((NACC, c2), jnp.float32)
    out = _make_edge_relu(c2)(_stack2(tp), _stack2(tq), src_s, dst_g, zeros)
    return _unstack(out)


def _tc(fn, out_shapes, *args):
    shapes = [jax.ShapeDtypeStruct(s, jnp.float32) for s in out_shapes]
    outs = pl.pallas_call(fn, out_shape=shapes)(*args)
    return outs


def _k_in(x_ref, w_ref, b_ref, o_ref):
    o_ref[...] = jnp.maximum(
        jnp.dot(x_ref[...], w_ref[...], preferred_element_type=jnp.float32)
        + b_ref[...], 0.0)


def _k_post1(r_ref, wd_ref, h_ref, ws_ref, b_ref, deg_ref, o_ref):
    t = jnp.dot(h_ref[...], ws_ref[...], preferred_element_type=jnp.float32)
    o_ref[...] = jnp.maximum(
        jnp.dot(r_ref[...], wd_ref[...], preferred_element_type=jnp.float32)
        + deg_ref[...] * (t + b_ref[...]), 0.0)


def _k_pq(h_ref, wd_ref, ba_ref, ws_ref, p_ref, q_ref):
    p_ref[...] = jnp.dot(h_ref[...], wd_ref[...],
                         preferred_element_type=jnp.float32) + ba_ref[...]
    q_ref[...] = jnp.dot(h_ref[...], ws_ref[...],
                         preferred_element_type=jnp.float32)


_RB = 1000


def _k_s1(p_ref, q_ref, di_ref, do_ref, s1_ref):
    i = pl.program_id(0)
    part = jnp.sum(di_ref[...] * p_ref[...] + do_ref[...] * q_ref[...],
                   axis=0, keepdims=True)

    @pl.when(i == 0)
    def _():
        s1_ref[...] = jnp.zeros_like(s1_ref)

    s1_ref[...] += part


def _k_s2c(p_ref, q_ref, r_ref, di_ref, do_ref, s1_ref, s2_ref):
    i = pl.program_id(0)
    mean = s1_ref[...] * (1.0 / E)
    q = q_ref[...]
    pc = p_ref[...] - mean
    rc = r_ref[...] - do_ref[...] * mean
    part = jnp.sum(di_ref[...] * pc * pc + do_ref[...] * q * q
                   + 2.0 * q * rc, axis=0, keepdims=True)

    @pl.when(i == 0)
    def _():
        s2_ref[...] = jnp.zeros_like(s2_ref)

    s2_ref[...] += part


def _k_apply(p_ref, q_ref, s1_ref, s2_ref, pt_ref, qt_ref):
    mean = s1_ref[...] * (1.0 / E)
    scale = lax.rsqrt(s2_ref[...] * (1.0 / E) + 1e-5)
    pt_ref[...] = (p_ref[...] - mean) * scale
    qt_ref[...] = q_ref[...] * scale


def _stats_scale(P, Q, R, deg_in, deg_out):
    C = P.shape[1]
    rb = pl.BlockSpec((_RB, C), lambda i: (i, 0))
    db = pl.BlockSpec((_RB, 1), lambda i: (i, 0))
    fb = pl.BlockSpec((1, C), lambda i: (0, 0))
    (S1,) = pl.pallas_call(
        _k_s1, grid=(N // _RB,),
        in_specs=[rb, rb, db, db], out_specs=[fb],
        out_shape=[jax.ShapeDtypeStruct((1, C), jnp.float32)],
    )(P, Q, deg_in, deg_out)
    (S2,) = pl.pallas_call(
        _k_s2c, grid=(N // _RB,),
        in_specs=[rb, rb, rb, db, db, fb], out_specs=[fb],
        out_shape=[jax.ShapeDtypeStruct((1, C), jnp.float32)],
    )(P, Q, R, deg_in, deg_out, S1)
    return pl.pallas_call(
        _k_apply, grid=(N // _RB,),
        in_specs=[rb, rb, fb, fb], out_specs=[rb, rb],
        out_shape=[jax.ShapeDtypeStruct((N, C), jnp.float32)] * 2,
    )(P, Q, S1, S2)


def _k_post2_relu(g_ref, wb_ref, bb_ref, deg_ref, o_ref):
    o_ref[...] = jnp.maximum(
        jnp.dot(g_ref[...], wb_ref[...], preferred_element_type=jnp.float32)
        + deg_ref[...] * bb_ref[...], 0.0)


def _k_post2(g_ref, wb_ref, bb_ref, deg_ref, o_ref):
    o_ref[...] = jnp.dot(g_ref[...], wb_ref[...],
                         preferred_element_type=jnp.float32) \
        + deg_ref[...] * bb_ref[...]


def _k_premul(h_ref, wd_ref, ws_ref, b_ref, deg_ref, skip_ref, u_ref, v_ref):
    u_ref[...] = jnp.dot(h_ref[...], wd_ref[...],
                         preferred_element_type=jnp.float32)
    t = jnp.dot(h_ref[...], ws_ref[...], preferred_element_type=jnp.float32)
    v_ref[...] = deg_ref[...] * (t + b_ref[...]) + skip_ref[...]


def _k_fuse_u(ru_ref, v_ref, wd_ref, ws_ref, b_ref, deg_ref, skip_ref,
              u_ref, v2_ref):
    h = jnp.maximum(ru_ref[...] + v_ref[...], 0.0)
    u_ref[...] = jnp.dot(h, wd_ref[...], preferred_element_type=jnp.float32)
    t = jnp.dot(h, ws_ref[...], preferred_element_type=jnp.float32)
    v2_ref[...] = deg_ref[...] * (t + b_ref[...]) + skip_ref[...]


def _k_out(ru_ref, v_ref, w_ref, b_ref, o_ref):
    h = jnp.maximum(ru_ref[...] + v_ref[...], 0.0)
    o_ref[...] = jnp.dot(h, w_ref[...],
                         preferred_element_type=jnp.float32) + b_ref[...]


def kernel(x, edge_index, W_in, b_in, W_d0, b_d0, W_d1, b_d1,
           W_l1a, b_l1a, W_l1b, b_l1b, W_l2a, b_l2a, W_l2b, b_l2b,
           W_u1, b_u1, W_u0, b_u0, W_out, b_out):
    src = edge_index[0].astype(jnp.int32)
    dst = edge_index[1].astype(jnp.int32)
    pad = E_PAD - E
    src_s = jnp.concatenate([src, jnp.full((pad,), N, jnp.int32)])
    dst_g = jnp.concatenate([dst, jnp.zeros((pad,), jnp.int32)])
    dst_s = jnp.concatenate([dst, jnp.full((pad,), N, jnp.int32)])
    src_g = jnp.concatenate([src, jnp.zeros((pad,), jnp.int32)])
    s128 = src_s.reshape(E_PAD // 128, 128)
    d128 = dst_g.reshape(E_PAD // 128, 128)
    ds128 = dst_s.reshape(E_PAD // 128, 128)
    sg128 = src_g.reshape(E_PAD // 128, 128)
    s64 = src_s.reshape(E_PAD // 64, 64)
    d64 = dst_g.reshape(E_PAD // 64, 64)

    ones_tab = jnp.ones((2 * N, 16), jnp.float32)
    zeros16 = jnp.zeros((NACC, 16), jnp.float32)
    spmv16 = _make_spmv(16)
    deg_out = spmv16(ones_tab, s128, d128, zeros16)[0, :, 0:1]
    deg_in = spmv16(ones_tab, ds128, sg128, zeros16)[0, :, 0:1]

    H = W_in.shape[1]

    def pl2_block(h, Wa, ba, Wb, bb, relu_out):
        F = h.shape[1]
        P, Q = _tc(_k_pq, [(N, Wa.shape[1])] * 2,
                   h, Wa[:F], ba.reshape(1, -1), Wa[F:])
        R = _spmv(P, s128, d128)
        Pt, Qt = _stats_scale(P, Q, R, deg_in, deg_out)
        G = _edge_relu_sum(Pt, Qt, s64, d64)
        post = _k_post2_relu if relu_out else _k_post2
        (h2,) = _tc(post, [(N, Wb.shape[1])], G, Wb, bb.reshape(1, -1),
                    deg_out)
        return h2

    (h0,) = _tc(_k_in, [(N, H)], x, W_in, b_in.reshape(1, -1))
    R0 = _spmv(h0, s128, d128)
    (h1,) = _tc(_k_post1, [(N, 2 * H)], R0, W_d0[:H], h0, W_d0[H:],
                b_d0.reshape(1, -1), deg_out)
    R1 = _spmv(h1, s128, d128)
    (h2,) = _tc(_k_post1, [(N, 4 * H)], R1, W_d1[:2 * H], h1, W_d1[2 * H:],
                b_d1.reshape(1, -1), deg_out)
    h3 = pl2_block(h2, W_l1a, b_l1a, W_l1b, b_l1b, relu_out=True)
    h4 = pl2_block(h3, W_l2a, b_l2a, W_l2b, b_l2b, relu_out=False)
    U1, V1 = _tc(_k_premul, [(N, 2 * H)] * 2, h4, W_u1[:4 * H],
                 W_u1[4 * H:], b_u1.reshape(1, -1), deg_out, h1)
    RU1 = _spmv(U1, s128, d128)
    U0, V0 = _tc(_k_fuse_u, [(N, H)] * 2, RU1, V1, W_u0[:2 * H],
                 W_u0[2 * H:], b_u0.reshape(1, -1), deg_out, h0)
    RU0 = _spmv(U0, s128, d128)
    (out,) = _tc(_k_out, [(N, W_out.shape[1])], RU0, V0, W_out,
                 b_out.reshape(1, -1))
    return out
